# Initial kernel scaffold; baseline (speedup 1.0000x reference)
import jax
import jax.numpy as jnp
from jax.experimental import pallas as pl


def kernel(x, edge_index, batch, W1, b1, W2, b2, W3, b3, Wg1, bg1, Wg2, bg2, Wf1, bf1, Wf2, bf2):
    raise NotImplementedError("write your pallas kernel here")


# speedup vs baseline: 2.3526x; 2.3526x over previous
import functools

import jax
import jax.numpy as jnp
from jax import lax
from jax.experimental import pallas as pl
from jax.experimental.pallas import tpu as pltpu


N = 100000
G = 128


_PREC = lax.Precision.HIGHEST


def _dense_stage_body(y_ref, xs_ref, dinv_ref, w_ref, b_ref, o_ref, *, relu):
    z = dinv_ref[...] * (y_ref[...] + xs_ref[...])
    acc = jnp.dot(z, w_ref[...], preferred_element_type=jnp.float32,
                  precision=_PREC)
    acc = acc + b_ref[...]
    if relu:
        acc = jnp.maximum(acc, 0.0)
    o_ref[...] = acc


def _dense_stage(y, xs, dinv, w, b, *, relu=True, block=2000):
    n, din = y.shape
    dout = w.shape[1]
    grid = (n // block,)
    return pl.pallas_call(
        functools.partial(_dense_stage_body, relu=relu),
        grid=grid,
        in_specs=[
            pl.BlockSpec((block, din), lambda i: (i, 0)),
            pl.BlockSpec((block, din), lambda i: (i, 0)),
            pl.BlockSpec((block, 1), lambda i: (i, 0)),
            pl.BlockSpec((din, dout), lambda i: (0, 0)),
            pl.BlockSpec((1, dout), lambda i: (0, 0)),
        ],
        out_specs=pl.BlockSpec((block, dout), lambda i: (i, 0)),
        out_shape=jax.ShapeDtypeStruct((n, dout), jnp.float32),
    )(y, xs, dinv, w, b)


def _head_body(g_ref, wg1_ref, bg1_ref, wg2_ref, bg2_ref, wf1_ref, bf1_ref,
               wf2_ref, bf2_ref, o_ref):
    g = g_ref[...]
    g = jnp.maximum(jnp.dot(g, wg1_ref[...], preferred_element_type=jnp.float32,
                            precision=_PREC) + bg1_ref[...], 0.0)
    g = jnp.maximum(jnp.dot(g, wg2_ref[...], preferred_element_type=jnp.float32,
                            precision=_PREC) + bg2_ref[...], 0.0)
    g = jnp.maximum(jnp.dot(g, wf1_ref[...], preferred_element_type=jnp.float32,
                            precision=_PREC) + bf1_ref[...], 0.0)
    o_ref[...] = (jnp.dot(g, wf2_ref[...], preferred_element_type=jnp.float32,
                          precision=_PREC) + bf2_ref[...])


def _head(g, Wg1, bg1, Wg2, bg2, Wf1, bf1, Wf2, bf2):
    return pl.pallas_call(
        _head_body,
        out_shape=jax.ShapeDtypeStruct((G, 1), jnp.float32),
    )(g, Wg1, bg1[None, :], Wg2, bg2[None, :], Wf1, bf1[None, :],
      Wf2, bf2[None, :])


def kernel(x, edge_index, batch, W1, b1, W2, b2, W3, b3,
           Wg1, bg1, Wg2, bg2, Wf1, bf1, Wf2, bf2):
    src = edge_index[0]
    dst = edge_index[1]

    deg = jnp.zeros((N,), jnp.float32).at[dst].add(1.0) + 1.0
    dinv = lax.rsqrt(deg)[:, None]

    def prop(xs):
        return jnp.zeros(xs.shape, jnp.float32).at[dst].add(xs[src])

    h = x
    for (W, b) in ((W1, b1), (W2, b2), (W3, b3)):
        xs = dinv * h
        y = prop(xs)
        h = _dense_stage(y, xs, dinv, W, b[None, :], relu=True)

    seg_max = jax.ops.segment_max(h, batch, num_segments=G)
    seg_sum = jax.ops.segment_sum(h, batch, num_segments=G)
    cnt = jax.ops.segment_sum(jnp.ones((N,), jnp.float32), batch,
                              num_segments=G)
    gmp = jnp.where(jnp.isfinite(seg_max), seg_max, 0.0)
    gap = seg_sum / jnp.maximum(cnt, 1.0)[:, None]
    g = jnp.concatenate([gmp, gap], axis=1)

    return _head(g, Wg1, bg1, Wg2, bg2, Wf1, bf1, Wf2, bf2)


# speedup vs baseline: 5.8853x; 2.5016x over previous
import functools

import jax
import jax.numpy as jnp
from jax import lax
from jax.experimental import pallas as pl
from jax.experimental.pallas import tpu as pltpu
from jax.experimental.pallas import tpu_sc as plsc


N = 100000
E = 1600000
G = 128

NC = 2
NS = 16
L = 16

EPT = E // NS
K = 800
NCHUNK = EPT // K
NPAD = 100096
NSLP = NPAD // NS
ZR = 368


@functools.lru_cache(maxsize=None)
def _sc_mesh():
    return plsc.VectorSubcoreMesh(core_axis_name="c", subcore_axis_name="s",
                                  num_cores=NC)


_SC_PARAMS = pltpu.CompilerParams(use_tc_tiling_on_sc=False)


@functools.lru_cache(maxsize=None)
def _make_prop(P):
    @functools.partial(
        pl.kernel,
        mesh=_sc_mesh(),
        compiler_params=_SC_PARAMS,
        out_type=jax.ShapeDtypeStruct((P * NPAD, L), jnp.float32),
        scratch_types=[
            pltpu.VMEM((K,), jnp.int32),
            pltpu.VMEM((K,), jnp.int32),
            pltpu.VMEM((K,), jnp.int32),
            pltpu.VMEM((K, L), jnp.float32),
            pltpu.VMEM((ZR, L), jnp.float32),
            pltpu.VMEM_SHARED((NPAD, L), jnp.float32),
            pltpu.SemaphoreType.DMA,
        ],
    )
    def prop(xs_hbm, src_hbm, dst_hbm, out_hbm,
             srcbuf, dstbuf, idxbuf, rows, zbuf, acc, sem):
        c = lax.axis_index("c")
        s = lax.axis_index("s")

        def zfill(i, _):
            zbuf[i] = jnp.zeros((L,), jnp.float32)
            return 0
        lax.fori_loop(0, ZR, zfill, 0)

        for pi in range((P + 1) // 2):
            p = pi * NC + c
            valid = p < P

            @pl.when(valid)
            def _zero():
                for j in range(NSLP // ZR):
                    pltpu.sync_copy(zbuf, acc.at[pl.ds(s * NSLP + j * ZR, ZR)])

            plsc.subcore_barrier()

            @pl.when(valid)
            def _sweep():
                def chunk(ci, _):
                    base = s * EPT + ci * K
                    pltpu.sync_copy(src_hbm.at[pl.ds(base, K)], srcbuf)
                    pltpu.sync_copy(dst_hbm.at[pl.ds(base, K)], dstbuf)

                    def idxstep(j, _):
                        idxbuf[pl.ds(j * L, L)] = srcbuf[pl.ds(j * L, L)] + p * N
                        return 0
                    lax.fori_loop(0, K // L, idxstep, 0)

                    pltpu.async_copy(xs_hbm.at[idxbuf], rows, sem).wait()
                    pltpu.sync_copy(rows, acc.at[dstbuf], add=True)
                    return 0
                lax.fori_loop(0, NCHUNK, chunk, 0)

            plsc.subcore_barrier()

            @pl.when(valid)
            def _writeout():
                pltpu.sync_copy(
                    acc.at[pl.ds(s * NSLP, NSLP)],
                    out_hbm.at[pl.ds(p * NPAD + s * NSLP, NSLP)])

            plsc.subcore_barrier()

    return prop


_EPC = E // (NC * NS)
KD = 400
_DCH = _EPC // KD


@functools.lru_cache(maxsize=None)
def _make_deg():
    @functools.partial(
        pl.kernel,
        mesh=_sc_mesh(),
        compiler_params=_SC_PARAMS,
        out_type=jax.ShapeDtypeStruct((NC * NPAD, L), jnp.float32),
        scratch_types=[
            pltpu.VMEM((KD,), jnp.int32),
            pltpu.VMEM((KD, L), jnp.float32),
            pltpu.VMEM((ZR, L), jnp.float32),
            pltpu.VMEM_SHARED((NPAD, L), jnp.float32),
        ],
    )
    def deg_kernel(dst_hbm, out_hbm, dstbuf, ones, zbuf, acc):
        c = lax.axis_index("c")
        s = lax.axis_index("s")

        def fill1(i, _):
            ones[i] = jnp.full((L,), 1.0, jnp.float32)
            return 0
        lax.fori_loop(0, KD, fill1, 0)

        def fill0(i, _):
            zbuf[i] = jnp.zeros((L,), jnp.float32)
            return 0
        lax.fori_loop(0, ZR, fill0, 0)

        for j in range(NSLP // ZR):
            pltpu.sync_copy(zbuf, acc.at[pl.ds(s * NSLP + j * ZR, ZR)])

        plsc.subcore_barrier()

        def chunk(ci, _):
            base = (c * NS + s) * _EPC + ci * KD
            pltpu.sync_copy(dst_hbm.at[pl.ds(base, KD)], dstbuf)
            pltpu.sync_copy(ones, acc.at[dstbuf], add=True)
            return 0
        lax.fori_loop(0, _DCH, chunk, 0)

        plsc.subcore_barrier()

        pltpu.sync_copy(acc.at[pl.ds(s * NSLP, NSLP)],
                        out_hbm.at[pl.ds(c * NPAD + s * NSLP, NSLP)])

    return deg_kernel


_PREC = lax.Precision.HIGHEST


def _prep_body(x_ref, d0_ref, d1_ref, xs_ref, dinv_ref):
    deg = d0_ref[...] + d1_ref[...] + 1.0
    dinv = lax.rsqrt(deg)
    xs_ref[...] = dinv * x_ref[...]
    dinv_ref[...] = dinv


def _prep_stage(x, deg0, deg1, *, block=2000):
    n, din = x.shape
    grid = (n // block,)
    return pl.pallas_call(
        _prep_body,
        grid=grid,
        in_specs=[
            pl.BlockSpec((block, din), lambda i: (i, 0)),
            pl.BlockSpec((block, 1), lambda i: (i, 0)),
            pl.BlockSpec((block, 1), lambda i: (i, 0)),
        ],
        out_specs=[
            pl.BlockSpec((block, din), lambda i: (i, 0)),
            pl.BlockSpec((block, 1), lambda i: (i, 0)),
        ],
        out_shape=[
            jax.ShapeDtypeStruct((n, din), jnp.float32),
            jax.ShapeDtypeStruct((n, 1), jnp.float32),
        ],
    )(x, deg0, deg1)


def _dense_stage_body(y_ref, xs_ref, dinv_ref, w_ref, b_ref, o_ref,
                      *, scale_out):
    dinv = dinv_ref[...]
    z = dinv * (y_ref[...] + xs_ref[...])
    acc = jnp.dot(z, w_ref[...], preferred_element_type=jnp.float32,
                  precision=_PREC)
    acc = jnp.maximum(acc + b_ref[...], 0.0)
    if scale_out:
        acc = dinv * acc
    o_ref[...] = acc


def _dense_stage(y, xs, dinv, w, b, *, scale_out, block=2000):
    n, din = y.shape
    dout = w.shape[1]
    grid = (n // block,)
    return pl.pallas_call(
        functools.partial(_dense_stage_body, scale_out=scale_out),
        grid=grid,
        in_specs=[
            pl.BlockSpec((block, din), lambda i: (i, 0)),
            pl.BlockSpec((block, din), lambda i: (i, 0)),
            pl.BlockSpec((block, 1), lambda i: (i, 0)),
            pl.BlockSpec((din, dout), lambda i: (0, 0)),
            pl.BlockSpec((1, dout), lambda i: (0, 0)),
        ],
        out_specs=pl.BlockSpec((block, dout), lambda i: (i, 0)),
        out_shape=jax.ShapeDtypeStruct((n, dout), jnp.float32),
    )(y, xs, dinv, w, b)


def _head_body(g_ref, wg1_ref, bg1_ref, wg2_ref, bg2_ref, wf1_ref, bf1_ref,
               wf2_ref, bf2_ref, o_ref):
    g = g_ref[...]
    g = jnp.maximum(jnp.dot(g, wg1_ref[...], preferred_element_type=jnp.float32,
                            precision=_PREC) + bg1_ref[...], 0.0)
    g = jnp.maximum(jnp.dot(g, wg2_ref[...], preferred_element_type=jnp.float32,
                            precision=_PREC) + bg2_ref[...], 0.0)
    g = jnp.maximum(jnp.dot(g, wf1_ref[...], preferred_element_type=jnp.float32,
                            precision=_PREC) + bf1_ref[...], 0.0)
    o_ref[...] = (jnp.dot(g, wf2_ref[...], preferred_element_type=jnp.float32,
                          precision=_PREC) + bf2_ref[...])


def _head(g, Wg1, bg1, Wg2, bg2, Wf1, bf1, Wf2, bf2):
    return pl.pallas_call(
        _head_body,
        out_shape=jax.ShapeDtypeStruct((G, 1), jnp.float32),
    )(g, Wg1, bg1[None, :], Wg2, bg2[None, :], Wf1, bf1[None, :],
      Wf2, bf2[None, :])


def _to_panels(xs_nat, P):
    n, d = xs_nat.shape
    xs = jnp.pad(xs_nat, ((0, 0), (0, P * L - d)))
    return jnp.transpose(xs.reshape(n, P, L), (1, 0, 2)).reshape(P * n, L)


def _from_panels(y_pm, P, d):
    y = y_pm.reshape(P, NPAD, L)[:, :N]
    return jnp.transpose(y, (1, 0, 2)).reshape(N, P * L)


def _pad_cols(a, dpad):
    return jnp.pad(a, ((0, 0), (0, dpad - a.shape[1])))


def _pad_w(w, dpad):
    return jnp.pad(w, ((0, dpad - w.shape[0]), (0, 0)))


def kernel(x, edge_index, batch, W1, b1, W2, b2, W3, b3,
           Wg1, bg1, Wg2, bg2, Wf1, bf1, Wf2, bf2):
    src = edge_index[0]
    dst = edge_index[1]

    degp = _make_deg()(dst)
    deg0 = degp[0:N, 0:1]
    deg1 = degp[NPAD:NPAD + N, 0:1]

    xs_n, dinv = _prep_stage(x, deg0, deg1)

    Ws = ((W1, b1, 3, True), (W2, b2, 6, True), (W3, b3, 11, False))
    for (W, b, P, scale) in Ws:
        y_nat = _from_panels(_make_prop(P)(_to_panels(xs_n, P), src, dst), P,
                             xs_n.shape[1])
        xs_n = _dense_stage(y_nat, _pad_cols(xs_n, P * L), dinv,
                            _pad_w(W, P * L), b[None, :], scale_out=scale)
    h = xs_n

    seg_max = jax.ops.segment_max(h, batch, num_segments=G)
    seg_sum = jax.ops.segment_sum(h, batch, num_segments=G)
    cnt = jax.ops.segment_sum(jnp.ones((N,), jnp.float32), batch,
                              num_segments=G)
    gmp = jnp.where(jnp.isfinite(seg_max), seg_max, 0.0)
    gap = seg_sum / jnp.maximum(cnt, 1.0)[:, None]
    g = jnp.concatenate([gmp, gap], axis=1)

    return _head(g, Wg1, bg1, Wg2, bg2, Wf1, bf1, Wf2, bf2)


# speedup vs baseline: 7.8776x; 1.3385x over previous
import functools

import jax
import jax.numpy as jnp
from jax import lax
from jax.experimental import pallas as pl
from jax.experimental.pallas import tpu as pltpu
from jax.experimental.pallas import tpu_sc as plsc


N = 100000
E = 1600000
G = 128

NC = 2
NS = 16
L = 16

EPT = E // NS
K = 800
NCHUNK = EPT // K
NPAD = 100096
NSLP = NPAD // NS
ZR = 368


@functools.lru_cache(maxsize=None)
def _sc_mesh():
    return plsc.VectorSubcoreMesh(core_axis_name="c", subcore_axis_name="s",
                                  num_cores=NC)


_SC_PARAMS = pltpu.CompilerParams(use_tc_tiling_on_sc=False)


@functools.lru_cache(maxsize=None)
def _make_prop(P):
    @functools.partial(
        pl.kernel,
        mesh=_sc_mesh(),
        compiler_params=_SC_PARAMS,
        out_type=jax.ShapeDtypeStruct((P * NPAD, L), jnp.float32),
        scratch_types=[
            pltpu.VMEM((K,), jnp.int32),
            pltpu.VMEM((K,), jnp.int32),
            pltpu.VMEM((K,), jnp.int32),
            pltpu.VMEM((K,), jnp.int32),
            pltpu.VMEM((K, L), jnp.float32),
            pltpu.VMEM((K, L), jnp.float32),
            pltpu.VMEM_SHARED((NPAD, L), jnp.float32),
            pltpu.SemaphoreType.DMA,
            pltpu.SemaphoreType.DMA,
            pltpu.SemaphoreType.DMA,
            pltpu.SemaphoreType.DMA,
            pltpu.SemaphoreType.DMA,
            pltpu.SemaphoreType.DMA,
        ],
    )
    def prop(xs_hbm, src_hbm, dst_hbm, out_hbm,
             src0, src1, dst0, dst1, rows0, rows1, acc,
             ss0, ss1, sd0, sd1, sg0, sg1):
        c = lax.axis_index("c")
        s = lax.axis_index("s")
        srcb = (src0, src1)
        dstb = (dst0, dst1)
        rows = (rows0, rows1)
        ssem = (ss0, ss1)
        dsem = (sd0, sd1)
        gsem = (sg0, sg1)

        def ebase(ci):
            return s * EPT + ci * K

        def issue_idx(ci, m):
            pltpu.async_copy(src_hbm.at[pl.ds(ebase(ci), K)], srcb[m], ssem[m])
            pltpu.async_copy(dst_hbm.at[pl.ds(ebase(ci), K)], dstb[m], dsem[m])

        def wait_src(m):
            pltpu.make_async_copy(src_hbm.at[pl.ds(0, K)], srcb[m],
                                  ssem[m]).wait()

        def wait_dst(m):
            pltpu.make_async_copy(dst_hbm.at[pl.ds(0, K)], dstb[m],
                                  dsem[m]).wait()

        def start_gather(m, p):
            def idxstep(j, _):
                srcb[m][pl.ds(j * L, L)] = srcb[m][pl.ds(j * L, L)] + p * N
                return 0
            lax.fori_loop(0, K // L, idxstep, 0)
            pltpu.async_copy(xs_hbm.at[srcb[m]], rows[m], gsem[m])

        def wait_gather(m):
            pltpu.make_async_copy(xs_hbm.at[srcb[m]], rows[m],
                                  gsem[m]).wait()

        for pi in range((P + 1) // 2):
            p = pi * NC + c
            valid = p < P

            @pl.when(valid)
            def _zero():
                def zfill(i, _):
                    rows0[i] = jnp.zeros((L,), jnp.float32)
                    return 0
                lax.fori_loop(0, K, zfill, 0)
                nfull = NSLP // K
                for j in range(nfull):
                    pltpu.sync_copy(rows0, acc.at[pl.ds(s * NSLP + j * K, K)])
                rem = NSLP - nfull * K
                if rem:
                    pltpu.sync_copy(rows0.at[pl.ds(0, rem)],
                                    acc.at[pl.ds(s * NSLP + nfull * K, rem)])

            plsc.subcore_barrier()

            @pl.when(valid)
            def _sweep():
                issue_idx(0, 0)
                issue_idx(1, 1)
                wait_src(0)
                start_gather(0, p)

                def process(ci, m):
                    o = 1 - m

                    @pl.when(ci + 1 < NCHUNK)
                    def _():
                        wait_src(o)
                        start_gather(o, p)

                    wait_gather(m)
                    wait_dst(m)
                    pltpu.sync_copy(rows[m], acc.at[dstb[m]], add=True)

                    @pl.when(ci + 2 < NCHUNK)
                    def _():
                        issue_idx(ci + 2, m)

                def pair(i, _):
                    process(2 * i, 0)

                    @pl.when(2 * i + 1 < NCHUNK)
                    def _():
                        process(2 * i + 1, 1)
                    return 0
                lax.fori_loop(0, (NCHUNK + 1) // 2, pair, 0)

            plsc.subcore_barrier()

            @pl.when(valid)
            def _writeout():
                pltpu.sync_copy(
                    acc.at[pl.ds(s * NSLP, NSLP)],
                    out_hbm.at[pl.ds(p * NPAD + s * NSLP, NSLP)])

            plsc.subcore_barrier()

    return prop


_EPC = E // (NC * NS)
KD = 400
_DCH = _EPC // KD


@functools.lru_cache(maxsize=None)
def _make_deg():
    @functools.partial(
        pl.kernel,
        mesh=_sc_mesh(),
        compiler_params=_SC_PARAMS,
        out_type=jax.ShapeDtypeStruct((NC * NPAD, L), jnp.float32),
        scratch_types=[
            pltpu.VMEM((KD,), jnp.int32),
            pltpu.VMEM((KD, L), jnp.float32),
            pltpu.VMEM((ZR, L), jnp.float32),
            pltpu.VMEM_SHARED((NPAD, L), jnp.float32),
        ],
    )
    def deg_kernel(dst_hbm, out_hbm, dstbuf, ones, zbuf, acc):
        c = lax.axis_index("c")
        s = lax.axis_index("s")

        def fill1(i, _):
            ones[i] = jnp.full((L,), 1.0, jnp.float32)
            return 0
        lax.fori_loop(0, KD, fill1, 0)

        def fill0(i, _):
            zbuf[i] = jnp.zeros((L,), jnp.float32)
            return 0
        lax.fori_loop(0, ZR, fill0, 0)

        for j in range(NSLP // ZR):
            pltpu.sync_copy(zbuf, acc.at[pl.ds(s * NSLP + j * ZR, ZR)])

        plsc.subcore_barrier()

        def chunk(ci, _):
            base = (c * NS + s) * _EPC + ci * KD
            pltpu.sync_copy(dst_hbm.at[pl.ds(base, KD)], dstbuf)
            pltpu.sync_copy(ones, acc.at[dstbuf], add=True)
            return 0
        lax.fori_loop(0, _DCH, chunk, 0)

        plsc.subcore_barrier()

        pltpu.sync_copy(acc.at[pl.ds(s * NSLP, NSLP)],
                        out_hbm.at[pl.ds(c * NPAD + s * NSLP, NSLP)])

    return deg_kernel


_PREC = lax.Precision.HIGHEST


def _prep_body(x_ref, d0_ref, d1_ref, xs_ref, dinv_ref):
    deg = d0_ref[...] + d1_ref[...] + 1.0
    dinv = lax.rsqrt(deg)
    xs_ref[...] = dinv * x_ref[...]
    dinv_ref[...] = dinv


def _prep_stage(x, deg0, deg1, *, block=2000):
    n, din = x.shape
    grid = (n // block,)
    return pl.pallas_call(
        _prep_body,
        grid=grid,
        in_specs=[
            pl.BlockSpec((block, din), lambda i: (i, 0)),
            pl.BlockSpec((block, 1), lambda i: (i, 0)),
            pl.BlockSpec((block, 1), lambda i: (i, 0)),
        ],
        out_specs=[
            pl.BlockSpec((block, din), lambda i: (i, 0)),
            pl.BlockSpec((block, 1), lambda i: (i, 0)),
        ],
        out_shape=[
            jax.ShapeDtypeStruct((n, din), jnp.float32),
            jax.ShapeDtypeStruct((n, 1), jnp.float32),
        ],
    )(x, deg0, deg1)


def _dense_stage_body(y_ref, xs_ref, dinv_ref, w_ref, b_ref, o_ref,
                      *, scale_out):
    dinv = dinv_ref[...]
    z = dinv * (y_ref[...] + xs_ref[...])
    acc = jnp.dot(z, w_ref[...], preferred_element_type=jnp.float32,
                  precision=_PREC)
    acc = jnp.maximum(acc + b_ref[...], 0.0)
    if scale_out:
        acc = dinv * acc
    o_ref[...] = acc


def _dense_stage(y, xs, dinv, w, b, *, scale_out, block=2000):
    n, din = y.shape
    dout = w.shape[1]
    grid = (n // block,)
    return pl.pallas_call(
        functools.partial(_dense_stage_body, scale_out=scale_out),
        grid=grid,
        in_specs=[
            pl.BlockSpec((block, din), lambda i: (i, 0)),
            pl.BlockSpec((block, din), lambda i: (i, 0)),
            pl.BlockSpec((block, 1), lambda i: (i, 0)),
            pl.BlockSpec((din, dout), lambda i: (0, 0)),
            pl.BlockSpec((1, dout), lambda i: (0, 0)),
        ],
        out_specs=pl.BlockSpec((block, dout), lambda i: (i, 0)),
        out_shape=jax.ShapeDtypeStruct((n, dout), jnp.float32),
    )(y, xs, dinv, w, b)


def _head_body(g_ref, wg1_ref, bg1_ref, wg2_ref, bg2_ref, wf1_ref, bf1_ref,
               wf2_ref, bf2_ref, o_ref):
    g = g_ref[...]
    g = jnp.maximum(jnp.dot(g, wg1_ref[...], preferred_element_type=jnp.float32,
                            precision=_PREC) + bg1_ref[...], 0.0)
    g = jnp.maximum(jnp.dot(g, wg2_ref[...], preferred_element_type=jnp.float32,
                            precision=_PREC) + bg2_ref[...], 0.0)
    g = jnp.maximum(jnp.dot(g, wf1_ref[...], preferred_element_type=jnp.float32,
                            precision=_PREC) + bf1_ref[...], 0.0)
    o_ref[...] = (jnp.dot(g, wf2_ref[...], preferred_element_type=jnp.float32,
                          precision=_PREC) + bf2_ref[...])


def _head(g, Wg1, bg1, Wg2, bg2, Wf1, bf1, Wf2, bf2):
    return pl.pallas_call(
        _head_body,
        out_shape=jax.ShapeDtypeStruct((G, 1), jnp.float32),
    )(g, Wg1, bg1[None, :], Wg2, bg2[None, :], Wf1, bf1[None, :],
      Wf2, bf2[None, :])


def _to_panels(xs_nat, P):
    n, d = xs_nat.shape
    xs = jnp.pad(xs_nat, ((0, 0), (0, P * L - d)))
    return jnp.transpose(xs.reshape(n, P, L), (1, 0, 2)).reshape(P * n, L)


def _from_panels(y_pm, P, d):
    y = y_pm.reshape(P, NPAD, L)[:, :N]
    return jnp.transpose(y, (1, 0, 2)).reshape(N, P * L)


def _pad_cols(a, dpad):
    return jnp.pad(a, ((0, 0), (0, dpad - a.shape[1])))


def _pad_w(w, dpad):
    return jnp.pad(w, ((0, dpad - w.shape[0]), (0, 0)))


def kernel(x, edge_index, batch, W1, b1, W2, b2, W3, b3,
           Wg1, bg1, Wg2, bg2, Wf1, bf1, Wf2, bf2):
    src = edge_index[0]
    dst = edge_index[1]

    degp = _make_deg()(dst)
    deg0 = degp[0:N, 0:1]
    deg1 = degp[NPAD:NPAD + N, 0:1]

    xs_n, dinv = _prep_stage(x, deg0, deg1)

    Ws = ((W1, b1, 3, True), (W2, b2, 6, True), (W3, b3, 11, False))
    for (W, b, P, scale) in Ws:
        y_nat = _from_panels(_make_prop(P)(_to_panels(xs_n, P), src, dst), P,
                             xs_n.shape[1])
        xs_n = _dense_stage(y_nat, _pad_cols(xs_n, P * L), dinv,
                            _pad_w(W, P * L), b[None, :], scale_out=scale)
    h = xs_n

    seg_max = jax.ops.segment_max(h, batch, num_segments=G)
    seg_sum = jax.ops.segment_sum(h, batch, num_segments=G)
    cnt = jax.ops.segment_sum(jnp.ones((N,), jnp.float32), batch,
                              num_segments=G)
    gmp = jnp.where(jnp.isfinite(seg_max), seg_max, 0.0)
    gap = seg_sum / jnp.maximum(cnt, 1.0)[:, None]
    g = jnp.concatenate([gmp, gap], axis=1)

    return _head(g, Wg1, bg1, Wg2, bg2, Wf1, bf1, Wf2, bf2)
